# Initial kernel scaffold; baseline (speedup 1.0000x reference)
import jax
import jax.numpy as jnp
from jax.experimental import pallas as pl


def kernel(x, W, b):
    raise NotImplementedError("write your pallas kernel here")


# speedup vs baseline: 1.9853x; 1.9853x over previous
import jax
import jax.numpy as jnp
from jax.experimental import pallas as pl
from jax.experimental.pallas import tpu as pltpu

_R = 10
_MAXR = 2 * _R - 1
_KBLK = 256


def _interact_kernel(xf_ref, xb_ref, w_ref, b_ref, xw_ref, xwt_ref, s_ref):
    k = pl.program_id(0)
    x = xf_ref[...]
    w = w_ref[...]

    @pl.when(k == 0)
    def _init():
        bvec = b_ref[...]
        xw_ref[...] = jnp.broadcast_to(bvec, xw_ref.shape)
        s_ref[...] = jnp.sum(x * bvec, axis=-1, keepdims=True)

    xk = xb_ref[...]
    p = jnp.dot(xk, w, preferred_element_type=jnp.float32)
    q = jax.lax.dot_general(x, w, (((1,), (1,)), ((), ())),
                            preferred_element_type=jnp.float32)
    xw_ref[...] += p
    xwt_ref[...] = q
    s_ref[...] += jnp.sum(p * x, axis=-1, keepdims=True)


def _interact(u, W, b2):
    B, D = u.shape
    grid = (D // _KBLK,)
    return pl.pallas_call(
        _interact_kernel,
        grid=grid,
        in_specs=[
            pl.BlockSpec((B, D), lambda k: (0, 0)),
            pl.BlockSpec((B, _KBLK), lambda k: (0, k)),
            pl.BlockSpec((_KBLK, D), lambda k: (k, 0)),
            pl.BlockSpec((1, D), lambda k: (0, 0)),
        ],
        out_specs=[
            pl.BlockSpec((B, D), lambda k: (0, 0)),
            pl.BlockSpec((B, _KBLK), lambda k: (0, k)),
            pl.BlockSpec((B, 1), lambda k: (0, 0)),
        ],
        out_shape=[
            jax.ShapeDtypeStruct((B, D), jnp.float32),
            jax.ShapeDtypeStruct((B, D), jnp.float32),
            jax.ShapeDtypeStruct((B, 1), jnp.float32),
        ],
    )(u, u, W, b2)


def _sampler_kernel(xw_ref, xwt_ref, x_ref, g_ref, rm_ref, sx_ref,
                    y_ref, lf_ref, idx_ref,
                    l_ref, d_ref, m_ref, S_ref, acc_ref):
    s = pl.program_id(0)

    @pl.when(s == 0)
    def _init():
        delta = 1.0 - 2.0 * x_ref[...]
        grad = xw_ref[...] + xwt_ref[...]
        l0 = delta * grad * 0.5
        l_ref[...] = l0
        d_ref[...] = delta
        m = jnp.max(jnp.abs(l0), axis=-1, keepdims=True)
        m_ref[...] = m
        S_ref[...] = jnp.sum(jnp.exp(l0 - m), axis=-1, keepdims=True)
        acc_ref[...] = jnp.zeros_like(acc_ref)

    l = l_ref[...]
    t = l + g_ref[0]
    tmax = jnp.max(t, axis=-1, keepdims=True)
    iota = jax.lax.broadcasted_iota(jnp.int32, t.shape, 1)
    big = jnp.int32(t.shape[-1])
    idx = jnp.min(jnp.where(t == tmax, iota, big), axis=-1, keepdims=True)
    onehot = iota == idx
    sel = jnp.sum(jnp.where(onehot, l, 0.0), axis=-1, keepdims=True)
    mask = rm_ref[0]
    m = m_ref[...]
    S = S_ref[...]
    acc_ref[...] += mask * (sel - (m + jnp.log(S)))
    do = onehot & (mask > 0.0)
    l_ref[...] = jnp.where(do, -l, l)
    d_ref[...] = jnp.where(do, -d_ref[...], d_ref[...])
    S_ref[...] = S + mask * (jnp.exp(-sel - m) - jnp.exp(sel - m))
    idx_ref[0] = idx

    @pl.when(s == _MAXR - 1)
    def _fin():
        y_ref[...] = (1.0 - d_ref[...]) * 0.5
        lf_ref[...] = acc_ref[...] + sx_ref[...]


def _sampler(xw, xwt, x, G, rmask_sb, sx):
    B, D = x.shape
    return pl.pallas_call(
        _sampler_kernel,
        grid=(_MAXR,),
        in_specs=[
            pl.BlockSpec((B, D), lambda s: (0, 0)),
            pl.BlockSpec((B, D), lambda s: (0, 0)),
            pl.BlockSpec((B, D), lambda s: (0, 0)),
            pl.BlockSpec((1, B, D), lambda s: (s, 0, 0)),
            pl.BlockSpec((1, B, 1), lambda s: (s, 0, 0)),
            pl.BlockSpec((B, 1), lambda s: (0, 0)),
        ],
        out_specs=[
            pl.BlockSpec((B, D), lambda s: (0, 0)),
            pl.BlockSpec((B, 1), lambda s: (0, 0)),
            pl.BlockSpec((1, B, 1), lambda s: (s, 0, 0)),
        ],
        out_shape=[
            jax.ShapeDtypeStruct((B, D), jnp.float32),
            jax.ShapeDtypeStruct((B, 1), jnp.float32),
            jax.ShapeDtypeStruct((_MAXR, B, 1), jnp.int32),
        ],
        scratch_shapes=[
            pltpu.VMEM((B, D), jnp.float32),
            pltpu.VMEM((B, D), jnp.float32),
            pltpu.VMEM((B, 1), jnp.float32),
            pltpu.VMEM((B, 1), jnp.float32),
            pltpu.VMEM((B, 1), jnp.float32),
        ],
    )(xw, xwt, x, G, rmask_sb, sx)


def _backward_kernel(yw_ref, ywt_ref, y_ref, x_ref, idx_ref, rm_ref,
                     sy_ref, lf_ref, u_ref, out_ref):
    y = y_ref[...]
    delta = 1.0 - 2.0 * y
    grad = yw_ref[...] + ywt_ref[...]
    l = delta * grad * 0.5
    m = jnp.max(jnp.abs(l), axis=-1, keepdims=True)
    S = jnp.sum(jnp.exp(l - m), axis=-1, keepdims=True)
    iota = jax.lax.broadcasted_iota(jnp.int32, l.shape, 1)
    acc = jnp.zeros_like(m)
    for s in range(_MAXR - 1, -1, -1):
        idx = idx_ref[s]
        onehot = iota == idx
        sel = jnp.sum(jnp.where(onehot, l, 0.0), axis=-1, keepdims=True)
        mask = rm_ref[:, s:s + 1]
        acc += mask * (sel - (m + jnp.log(S)))
        if s > 0:
            do = onehot & (mask > 0.0)
            l = jnp.where(do, -l, l)
            S = S + mask * (jnp.exp(-sel - m) - jnp.exp(sel - m))
    log_backwd = acc + sy_ref[...]
    log_acc = log_backwd - lf_ref[...]
    accept = (jnp.exp(log_acc) >= u_ref[...]).astype(jnp.float32)
    out_ref[...] = y * accept + (1.0 - accept) * x_ref[...]


def _backward(yw, ywt, y, x, idxarr, rmask, sy, lf, u):
    B, D = x.shape
    return pl.pallas_call(
        _backward_kernel,
        out_shape=jax.ShapeDtypeStruct((B, D), jnp.float32),
    )(yw, ywt, y, x, idxarr, rmask, sy, lf, u)


def kernel(x, W, b):
    B, D = x.shape
    key = jax.random.key(42)
    k_r, k_loop, k_acc = jax.random.split(key, 3)
    radius = jax.random.randint(k_r, (B, 1), 1, 2 * _R)
    r_mask = (jnp.arange(_MAXR)[None, :] < radius).astype(jnp.float32)
    G = jnp.stack([
        jax.random.gumbel(jax.random.fold_in(k_loop, s), (B, D), jnp.float32)
        for s in range(_MAXR)
    ])
    u = jax.random.uniform(k_acc, (B,)).reshape(B, 1)
    b2 = b.reshape(1, D)
    rmask_sb = r_mask.T.reshape(_MAXR, B, 1)

    xw, xwt, sx = _interact(x, W, b2)
    y, lf, idxarr = _sampler(xw, xwt, x, G, rmask_sb, sx)
    yw, ywt, sy = _interact(y, W, b2)
    return _backward(yw, ywt, y, x, idxarr, r_mask, sy, lf, u)


# speedup vs baseline: 3.3013x; 1.6628x over previous
import jax
import jax.numpy as jnp
from jax.experimental import pallas as pl
from jax.experimental.pallas import tpu as pltpu

_R = 10
_MAXR = 2 * _R - 1
_KBLK = 256


def _interact_kernel(xf_ref, xb_ref, w_ref, b_ref, xw_ref, xwt_ref, s_ref):
    k = pl.program_id(0)
    x = xf_ref[...]
    w = w_ref[...]

    @pl.when(k == 0)
    def _init():
        bvec = b_ref[...]
        xw_ref[...] = jnp.broadcast_to(bvec, xw_ref.shape)
        s_ref[...] = jnp.sum(x * bvec, axis=-1, keepdims=True)

    xk = xb_ref[...]
    p = jnp.dot(xk, w, preferred_element_type=jnp.float32)
    q = jax.lax.dot_general(x, w, (((1,), (1,)), ((), ())),
                            preferred_element_type=jnp.float32)
    xw_ref[...] += p
    xwt_ref[...] = q
    s_ref[...] += jnp.sum(p * x, axis=-1, keepdims=True)


def _interact(u, W, b2):
    B, D = u.shape
    grid = (D // _KBLK,)
    return pl.pallas_call(
        _interact_kernel,
        grid=grid,
        in_specs=[
            pl.BlockSpec((B, D), lambda k: (0, 0)),
            pl.BlockSpec((B, _KBLK), lambda k: (0, k)),
            pl.BlockSpec((_KBLK, D), lambda k: (k, 0)),
            pl.BlockSpec((1, D), lambda k: (0, 0)),
        ],
        out_specs=[
            pl.BlockSpec((B, D), lambda k: (0, 0)),
            pl.BlockSpec((B, _KBLK), lambda k: (0, k)),
            pl.BlockSpec((B, 1), lambda k: (0, 0)),
        ],
        out_shape=[
            jax.ShapeDtypeStruct((B, D), jnp.float32),
            jax.ShapeDtypeStruct((B, D), jnp.float32),
            jax.ShapeDtypeStruct((B, 1), jnp.float32),
        ],
    )(u, u, W, b2)


def _sampler_kernel(xw_ref, xwt_ref, x_ref, g_ref, rm_ref, sx_ref,
                    y_ref, lf_ref, idx_ref,
                    l_ref, d_ref, m_ref, S_ref, acc_ref):
    s = pl.program_id(0)

    @pl.when(s == 0)
    def _init():
        delta = 1.0 - 2.0 * x_ref[...]
        grad = xw_ref[...] + xwt_ref[...]
        l0 = delta * grad * 0.5
        l_ref[...] = l0
        d_ref[...] = delta
        m = jnp.max(jnp.abs(l0), axis=-1, keepdims=True)
        m_ref[...] = m
        S_ref[...] = jnp.sum(jnp.exp(l0 - m), axis=-1, keepdims=True)
        acc_ref[...] = jnp.zeros_like(acc_ref)

    l = l_ref[...]
    t = l + g_ref[0]
    tmax = jnp.max(t, axis=-1, keepdims=True)
    iota = jax.lax.broadcasted_iota(jnp.int32, t.shape, 1)
    big = jnp.int32(t.shape[-1])
    idx = jnp.min(jnp.where(t == tmax, iota, big), axis=-1, keepdims=True)
    onehot = iota == idx
    sel = jnp.sum(jnp.where(onehot, l, 0.0), axis=-1, keepdims=True)
    mask = rm_ref[0]
    m = m_ref[...]
    S = S_ref[...]
    acc_ref[...] += mask * (sel - (m + jnp.log(S)))
    do = onehot & (mask > 0.0)
    l_ref[...] = jnp.where(do, -l, l)
    d_ref[...] = jnp.where(do, -d_ref[...], d_ref[...])
    S_ref[...] = S + mask * (jnp.exp(-sel - m) - jnp.exp(sel - m))
    idx_ref[0] = idx

    @pl.when(s == _MAXR - 1)
    def _fin():
        y_ref[...] = (1.0 - d_ref[...]) * 0.5
        lf_ref[...] = acc_ref[...] + sx_ref[...]


def _sampler(xw, xwt, x, G, rmask_sb, sx):
    B, D = x.shape
    return pl.pallas_call(
        _sampler_kernel,
        grid=(_MAXR,),
        in_specs=[
            pl.BlockSpec((B, D), lambda s: (0, 0)),
            pl.BlockSpec((B, D), lambda s: (0, 0)),
            pl.BlockSpec((B, D), lambda s: (0, 0)),
            pl.BlockSpec((1, B, D), lambda s: (s, 0, 0)),
            pl.BlockSpec((1, B, 1), lambda s: (s, 0, 0)),
            pl.BlockSpec((B, 1), lambda s: (0, 0)),
        ],
        out_specs=[
            pl.BlockSpec((B, D), lambda s: (0, 0)),
            pl.BlockSpec((B, 1), lambda s: (0, 0)),
            pl.BlockSpec((1, B, 1), lambda s: (s, 0, 0)),
        ],
        out_shape=[
            jax.ShapeDtypeStruct((B, D), jnp.float32),
            jax.ShapeDtypeStruct((B, 1), jnp.float32),
            jax.ShapeDtypeStruct((_MAXR, B, 1), jnp.int32),
        ],
        scratch_shapes=[
            pltpu.VMEM((B, D), jnp.float32),
            pltpu.VMEM((B, D), jnp.float32),
            pltpu.VMEM((B, 1), jnp.float32),
            pltpu.VMEM((B, 1), jnp.float32),
            pltpu.VMEM((B, 1), jnp.float32),
        ],
    )(xw, xwt, x, G, rmask_sb, sx)


def _backward_kernel(yw_ref, ywt_ref, y_ref, x_ref, idx_ref, rm_ref,
                     sy_ref, lf_ref, u_ref, out_ref):
    y = y_ref[...]
    delta = 1.0 - 2.0 * y
    grad = yw_ref[...] + ywt_ref[...]
    l = delta * grad * 0.5
    m = jnp.max(jnp.abs(l), axis=-1, keepdims=True)
    S = jnp.sum(jnp.exp(l - m), axis=-1, keepdims=True)
    iota = jax.lax.broadcasted_iota(jnp.int32, l.shape, 1)
    acc = jnp.zeros_like(m)
    for s in range(_MAXR - 1, -1, -1):
        idx = idx_ref[s]
        onehot = iota == idx
        sel = jnp.sum(jnp.where(onehot, l, 0.0), axis=-1, keepdims=True)
        mask = rm_ref[:, s:s + 1]
        acc += mask * (sel - (m + jnp.log(S)))
        if s > 0:
            do = onehot & (mask > 0.0)
            l = jnp.where(do, -l, l)
            S = S + mask * (jnp.exp(-sel - m) - jnp.exp(sel - m))
    log_backwd = acc + sy_ref[...]
    log_acc = log_backwd - lf_ref[...]
    accept = (jnp.exp(log_acc) >= u_ref[...]).astype(jnp.float32)
    out_ref[...] = y * accept + (1.0 - accept) * x_ref[...]


def _backward(yw, ywt, y, x, idxarr, rmask, sy, lf, u):
    B, D = x.shape
    return pl.pallas_call(
        _backward_kernel,
        out_shape=jax.ShapeDtypeStruct((B, D), jnp.float32),
    )(yw, ywt, y, x, idxarr, rmask, sy, lf, u)


def kernel(x, W, b):
    B, D = x.shape
    key = jax.random.key(42)
    k_r, k_loop, k_acc = jax.random.split(key, 3)
    radius = jax.random.randint(k_r, (B, 1), 1, 2 * _R)
    r_mask = (jnp.arange(_MAXR)[None, :] < radius).astype(jnp.float32)
    G = jax.vmap(lambda s: jax.random.gumbel(
        jax.random.fold_in(k_loop, s), (B, D), jnp.float32))(jnp.arange(_MAXR))
    u = jax.random.uniform(k_acc, (B,)).reshape(B, 1)
    b2 = b.reshape(1, D)
    rmask_sb = r_mask.T.reshape(_MAXR, B, 1)

    xw, xwt, sx = _interact(x, W, b2)
    y, lf, idxarr = _sampler(xw, xwt, x, G, rmask_sb, sx)
    yw, ywt, sy = _interact(y, W, b2)
    return _backward(yw, ywt, y, x, idxarr, r_mask, sy, lf, u)
